# baseline (speedup 1.0000x reference)
import functools

import jax
import jax.numpy as jnp
from jax import lax
from jax.experimental import pallas as pl
from jax.experimental.pallas import tpu as pltpu
from jax.experimental.pallas import tpu_sc as plsc

NC, NS, L = 2, 16, 16
NW = NC * NS
D = 128
DP = D // 2
CHUNK = 128
NBUF = 5
S = 2


def _make_kernel(n_rows):
    rows_per_w = n_rows // NW
    n_chunks = rows_per_w // CHUNK
    n_epochs = n_chunks // NBUF
    mesh = plsc.VectorSubcoreMesh(core_axis_name="c", subcore_axis_name="s")

    @functools.partial(
        pl.kernel,
        out_type=jax.ShapeDtypeStruct((n_rows, DP), jnp.int32),
        mesh=mesh,
        scratch_types=[
            pltpu.VMEM((n_chunks, CHUNK), jnp.int32),
            [pltpu.VMEM((CHUNK, DP), jnp.int32)] * NBUF,
            [pltpu.SemaphoreType.DMA] * NBUF,
            [pltpu.SemaphoreType.DMA] * NBUF,
        ],
        compiler_params=pltpu.CompilerParams(use_tc_tiling_on_sc=False),
    )
    def emb_kernel(idx_hbm, table_hbm, out_hbm, idx_v, bufs, gsems, ssems):
        wid = lax.axis_index("s") * NC + lax.axis_index("c")
        base = wid * rows_per_w
        pltpu.sync_copy(idx_hbm.at[wid], idx_v)

        def gather(c, b):
            return pltpu.make_async_copy(
                table_hbm.at[idx_v.at[c]], bufs[b], gsems[b])

        def store(c, b):
            return pltpu.make_async_copy(
                bufs[b], out_hbm.at[pl.ds(base + c * CHUNK, CHUNK)], ssems[b])

        def block(t, b, retire, refill):
            c = t * NBUF + b
            gather(c, b).wait()
            store(c, b).start()
            if retire:
                bp = (b - S) % NBUF
                cp = c - S
                store(cp, bp).wait()
                if refill:
                    gather(cp + NBUF, bp).start()

        for b in range(NBUF):
            gather(b, b).start()

        for b in range(NBUF):
            block(0, b, retire=(b >= S), refill=True)

        @pl.loop(1, n_epochs - 1)
        def _epoch(t):
            for b in range(NBUF):
                block(t, b, retire=True, refill=True)

        for b in range(NBUF):
            block(n_epochs - 1, b, retire=True, refill=(b < S))
        for i in range(S):
            c = n_chunks - S + i
            store(c, c % NBUF).wait()

    return emb_kernel


def kernel(x, table):
    b, s = x.shape
    n_rows = b * s
    idx3 = x.reshape(NW, n_rows // (NW * CHUNK), CHUNK)
    packed = lax.bitcast_convert_type(
        table.astype(jnp.bfloat16).reshape(-1, DP, 2), jnp.int32)
    out_packed = _make_kernel(n_rows)(idx3, packed)
    out = lax.bitcast_convert_type(out_packed, jnp.bfloat16)
    return out.reshape(b, s, D).astype(jnp.float32)


# speedup vs baseline: 2.5099x; 2.5099x over previous
import functools

import jax
import jax.numpy as jnp
from jax import lax
from jax.experimental import pallas as pl
from jax.experimental.pallas import tpu as pltpu
from jax.experimental.pallas import tpu_sc as plsc

NC, NS, L = 2, 16, 16
NW = NC * NS
D = 128
DP = D // 2
CHUNK = 128
NBUF = 5
S = 2


def _make_kernel(n_rows):
    rows_per_w = n_rows // NW
    n_chunks = rows_per_w // CHUNK
    n_epochs = n_chunks // NBUF
    mesh = plsc.VectorSubcoreMesh(core_axis_name="c", subcore_axis_name="s")

    @functools.partial(
        pl.kernel,
        out_type=jax.ShapeDtypeStruct((n_rows, DP), jnp.int32),
        mesh=mesh,
        scratch_types=[
            pltpu.VMEM((n_chunks, CHUNK), jnp.int32),
            [pltpu.VMEM((CHUNK, DP), jnp.int32)] * NBUF,
            [pltpu.SemaphoreType.DMA] * NBUF,
            [pltpu.SemaphoreType.DMA] * NBUF,
        ],
        compiler_params=pltpu.CompilerParams(use_tc_tiling_on_sc=False),
    )
    def emb_kernel(idx_hbm, table_hbm, out_hbm, idx_v, bufs, gsems, ssems):
        wid = lax.axis_index("s") * NC + lax.axis_index("c")
        base = wid * rows_per_w
        pltpu.sync_copy(idx_hbm.at[wid], idx_v)

        def gather(c, b):
            return pltpu.make_async_copy(
                table_hbm.at[idx_v.at[c]], bufs[b], gsems[b])

        def store(c, b):
            return pltpu.make_async_copy(
                bufs[b], out_hbm.at[pl.ds(base + c * CHUNK, CHUNK)], ssems[b])

        def block(t, b, retire, refill):
            c = t * NBUF + b
            gather(c, b).wait()
            store(c, b).start()
            if retire:
                bp = (b - S) % NBUF
                cp = c - S
                store(cp, bp).wait()
                if refill:
                    gather(cp + NBUF, bp).start()

        for b in range(NBUF):
            gather(b, b).start()

        for b in range(NBUF):
            block(0, b, retire=(b >= S), refill=True)

        @pl.loop(1, n_epochs - 1)
        def _epoch(t):
            for b in range(NBUF):
                block(t, b, retire=True, refill=True)

        for b in range(NBUF):
            block(n_epochs - 1, b, retire=True, refill=(b < S))
        for i in range(S):
            c = n_chunks - S + i
            store(c, c % NBUF).wait()

    return emb_kernel


def kernel(x, table):
    b, s = x.shape
    n_rows = b * s
    idx3 = x.reshape(NW, n_rows // (NW * CHUNK), CHUNK)
    tb = table.astype(jnp.bfloat16)
    pairs = jnp.stack([tb[:, :DP], tb[:, DP:]], axis=-1)
    packed = lax.bitcast_convert_type(pairs, jnp.int32)
    out_packed = _make_kernel(n_rows)(idx3, packed)
    lo = lax.bitcast_convert_type(out_packed << 16, jnp.float32)
    hi = lax.bitcast_convert_type(
        out_packed & jnp.int32(-65536), jnp.float32)
    return jnp.concatenate([lo, hi], axis=-1).reshape(b, s, D)


# speedup vs baseline: 11.0929x; 4.4197x over previous
import functools

import jax
import jax.numpy as jnp
from jax import lax
from jax.experimental import pallas as pl
from jax.experimental.pallas import tpu as pltpu
from jax.experimental.pallas import tpu_sc as plsc

NC, NS, L = 2, 16, 16
NW = NC * NS
D = 128
CHUNK = 128
NBUF = 5
S = 2


def _make_kernel(n_rows):
    rows_per_w = n_rows // NW
    n_chunks = rows_per_w // CHUNK
    n_epochs = n_chunks // NBUF
    mesh = plsc.VectorSubcoreMesh(core_axis_name="c", subcore_axis_name="s")

    @functools.partial(
        pl.kernel,
        out_type=jax.ShapeDtypeStruct((n_rows, D), jnp.float32),
        mesh=mesh,
        scratch_types=[
            pltpu.VMEM((n_chunks, CHUNK), jnp.int32),
            [pltpu.VMEM((CHUNK, D), jnp.float32)] * NBUF,
            [pltpu.SemaphoreType.DMA] * NBUF,
            [pltpu.SemaphoreType.DMA] * NBUF,
        ],
    )
    def emb_kernel(idx_hbm, table_hbm, out_hbm, idx_v, bufs, gsems, ssems):
        wid = lax.axis_index("s") * NC + lax.axis_index("c")
        base = wid * rows_per_w
        pltpu.sync_copy(idx_hbm.at[wid], idx_v)

        def gather(c, b):
            return pltpu.make_async_copy(
                table_hbm.at[idx_v.at[c]], bufs[b], gsems[b])

        def store(c, b):
            return pltpu.make_async_copy(
                bufs[b], out_hbm.at[pl.ds(base + c * CHUNK, CHUNK)], ssems[b])

        def block(t, b, retire, refill):
            c = t * NBUF + b
            gather(c, b).wait()
            store(c, b).start()
            if retire:
                bp = (b - S) % NBUF
                cp = c - S
                store(cp, bp).wait()
                if refill:
                    gather(cp + NBUF, bp).start()

        for b in range(NBUF):
            gather(b, b).start()

        for b in range(NBUF):
            block(0, b, retire=(b >= S), refill=True)

        @pl.loop(1, n_epochs - 1)
        def _epoch(t):
            for b in range(NBUF):
                block(t, b, retire=True, refill=True)

        for b in range(NBUF):
            block(n_epochs - 1, b, retire=True, refill=(b < S))
        for i in range(S):
            c = n_chunks - S + i
            store(c, c % NBUF).wait()

    return emb_kernel


def kernel(x, table):
    b, s = x.shape
    n_rows = b * s
    idx3 = x.reshape(NW, n_rows // (NW * CHUNK), CHUNK)
    out = _make_kernel(n_rows)(idx3, table)
    return out.reshape(b, s, D)
